# Initial kernel scaffold; baseline (speedup 1.0000x reference)
import jax
import jax.numpy as jnp
from jax.experimental import pallas as pl


def kernel(x, pos, knn_idx, W_first, b_first, gamma_first, beta_first, W_mid0, b_mid0, gamma_mid0, beta_mid0, W_last, b_last, gamma_last, beta_last, W_out, b_out):
    raise NotImplementedError("write your pallas kernel here")


# speedup vs baseline: 10.8961x; 10.8961x over previous
import functools

import jax
import jax.numpy as jnp
from jax import lax
from jax.experimental import pallas as pl
from jax.experimental.pallas import tpu as pltpu
from jax.experimental.pallas import tpu_sc as plsc

_PC = pl.pallas_call

_NBA = 1000
_NBC = 200
_EPS = 1e-5


def _expand(t, k):
    nb, c = t.shape
    return jnp.broadcast_to(t[:, None, :], (nb, k, c)).reshape(nb * k, c)


def _proj_body(xt_ref, post_ref, w1t_ref, wpt_ref, w2t_ref, w3t_ref, wot_ref,
               bc_ref, p_ref, c2_ref, c3_ref, co_ref):
    xt = xt_ref[...]
    f32 = jnp.float32
    p_ref[...] = (jnp.dot(xt, w1t_ref[...], preferred_element_type=f32)
                  + jnp.dot(post_ref[...], wpt_ref[...],
                            preferred_element_type=f32))
    c2_ref[...] = (jnp.dot(xt, w2t_ref[...], preferred_element_type=f32)
                   + bc_ref[0:1, :16])
    c3_ref[...] = (jnp.dot(xt, w3t_ref[...], preferred_element_type=f32)
                   + bc_ref[1:2, :16])
    co_ref[...] = (jnp.dot(xt, wot_ref[...], preferred_element_type=f32)
                   + bc_ref[2:3, :])


def _node_tables_call(xt, post, w1t, wpt, w2t, w3t, wot, bc):
    m, c = xt.shape
    dim = wot.shape[1]
    grid = (m // _NBA,)
    blk = lambda i: (i, 0)
    cst = lambda i: (0, 0)
    f32 = jnp.float32
    return _PC(
        _proj_body,
        grid=grid,
        in_specs=[
            pl.BlockSpec((_NBA, c), blk),
            pl.BlockSpec((_NBA, 8), blk),
            pl.BlockSpec((c, 16), cst),
            pl.BlockSpec((8, 16), cst),
            pl.BlockSpec((c, 16), cst),
            pl.BlockSpec((c, 16), cst),
            pl.BlockSpec((c, dim), cst),
            pl.BlockSpec((8, dim), cst),
        ],
        out_specs=[
            pl.BlockSpec((_NBA, 16), blk),
            pl.BlockSpec((_NBA, 16), blk),
            pl.BlockSpec((_NBA, 16), blk),
            pl.BlockSpec((_NBA, dim), blk),
        ],
        out_shape=[
            jax.ShapeDtypeStruct((m, 16), f32),
            jax.ShapeDtypeStruct((m, 16), f32),
            jax.ShapeDtypeStruct((m, 16), f32),
            jax.ShapeDtypeStruct((m, dim), f32),
        ],
    )(xt, post, w1t, wpt, w2t, w3t, wot, bc)


def _sc_gather(table, idx_flat):
    e_total = idx_flat.shape[0]
    info = plsc.get_sparse_core_info()
    nc, ns = info.num_cores, info.num_subcores
    nw = nc * ns
    ew = e_total // nw
    ch = 80
    nch = ew // ch
    mesh = plsc.VectorSubcoreMesh(core_axis_name="c", subcore_axis_name="s")

    @functools.partial(
        pl.kernel,
        out_type=jax.ShapeDtypeStruct((e_total, 16), jnp.float32),
        mesh=mesh,
        scratch_types=[
            pltpu.VMEM((ch,), jnp.int32),
            pltpu.VMEM((ch, 16), jnp.float32),
            pltpu.SemaphoreType.DMA,
        ],
        compiler_params=pltpu.CompilerParams(use_tc_tiling_on_sc=False),
    )
    def k(table_hbm, idx_hbm, out_hbm, idx_v, rows_v, sem):
        wid = lax.axis_index("s") * nc + lax.axis_index("c")
        base = wid * ew

        def body(j, carry):
            off = base + j * ch
            pltpu.sync_copy(idx_hbm.at[pl.ds(off, ch)], idx_v)
            pltpu.async_copy(table_hbm.at[idx_v], rows_v, sem).wait()
            pltpu.sync_copy(rows_v, out_hbm.at[pl.ds(off, ch)])
            return carry

        lax.fori_loop(0, nch, body, 0)

    return k(table, idx_flat)


def _stats1_body(g_ref, p_ref, c_ref, o_ref):
    k = g_ref.shape[0] // p_ref.shape[0]
    y1p = g_ref[...] - _expand(p_ref[...], k) + c_ref[0:1, :]

    @pl.when(pl.program_id(0) == 0)
    def _():
        o_ref[...] = jnp.zeros_like(o_ref)

    o_ref[0:1, :] += jnp.sum(y1p, axis=0, keepdims=True)
    o_ref[1:2, :] += jnp.sum(y1p * y1p, axis=0, keepdims=True)


def _stats2_body(g_ref, p_ref, c2_ref, w2_ref, c_ref, o_ref):
    k = g_ref.shape[0] // p_ref.shape[0]
    y1p = g_ref[...] - _expand(p_ref[...], k) + c_ref[0:1, :]
    y1 = jnp.maximum(c_ref[1:2, :] * y1p + c_ref[2:3, :], 0.0)
    y2p = (jnp.dot(y1, w2_ref[...], preferred_element_type=jnp.float32)
           + _expand(c2_ref[...], k))

    @pl.when(pl.program_id(0) == 0)
    def _():
        o_ref[...] = jnp.zeros_like(o_ref)

    o_ref[0:1, :] += jnp.sum(y2p, axis=0, keepdims=True)
    o_ref[1:2, :] += jnp.sum(y2p * y2p, axis=0, keepdims=True)


def _stats3_body(g_ref, p_ref, c2_ref, c3_ref, w2_ref, w3_ref, c_ref, o_ref):
    k = g_ref.shape[0] // p_ref.shape[0]
    y1p = g_ref[...] - _expand(p_ref[...], k) + c_ref[0:1, :]
    y1 = jnp.maximum(c_ref[1:2, :] * y1p + c_ref[2:3, :], 0.0)
    y2p = (jnp.dot(y1, w2_ref[...], preferred_element_type=jnp.float32)
           + _expand(c2_ref[...], k))
    y2 = jnp.maximum(c_ref[3:4, :] * y2p + c_ref[4:5, :], 0.0)
    u2 = jnp.concatenate([y2, y1], axis=1)
    y3p = (jnp.dot(u2, w3_ref[...], preferred_element_type=jnp.float32)
           + _expand(c3_ref[...], k))

    @pl.when(pl.program_id(0) == 0)
    def _():
        o_ref[...] = jnp.zeros_like(o_ref)

    o_ref[0:1, :] += jnp.sum(y3p, axis=0, keepdims=True)
    o_ref[1:2, :] += jnp.sum(y3p * y3p, axis=0, keepdims=True)


def _final_body(g_ref, p_ref, c2_ref, c3_ref, co_ref, w2_ref, w3_ref, wo_ref,
                c_ref, o_ref):
    nb = p_ref.shape[0]
    k = g_ref.shape[0] // nb
    y1p = g_ref[...] - _expand(p_ref[...], k) + c_ref[0:1, :]
    y1 = jnp.maximum(c_ref[1:2, :] * y1p + c_ref[2:3, :], 0.0)
    y2p = (jnp.dot(y1, w2_ref[...], preferred_element_type=jnp.float32)
           + _expand(c2_ref[...], k))
    y2 = jnp.maximum(c_ref[3:4, :] * y2p + c_ref[4:5, :], 0.0)
    u2 = jnp.concatenate([y2, y1], axis=1)
    y3p = (jnp.dot(u2, w3_ref[...], preferred_element_type=jnp.float32)
           + _expand(c3_ref[...], k))
    y3 = jnp.maximum(c_ref[5:6, :] * y3p + c_ref[6:7, :], 0.0)
    u = jnp.concatenate([y3, y2, y1], axis=1)
    o = jnp.dot(u, wo_ref[...], preferred_element_type=jnp.float32)
    dim = o.shape[1]
    o3 = o.reshape(nb, k, dim)
    m = o3[:, 0, :]
    for kk in range(1, k):
        m = jnp.maximum(m, o3[:, kk, :])
    o_ref[...] = m + co_ref[...]


def _edge_pass(body, ins, n_nodes, k, dim=None):
    e_total = n_nodes * k
    eblk = _NBC * k
    grid = (e_total // eblk,)
    blk_e = lambda i: (i, 0)
    blk_n = lambda i: (i, 0)
    cst = lambda i: (0, 0)
    f32 = jnp.float32
    in_specs = []
    for a in ins:
        if a.shape[0] == e_total:
            in_specs.append(pl.BlockSpec((eblk, a.shape[1]), blk_e))
        elif a.shape[0] == n_nodes:
            in_specs.append(pl.BlockSpec((_NBC, a.shape[1]), blk_n))
        else:
            in_specs.append(pl.BlockSpec(a.shape, cst))
    if dim is None:
        out_spec = pl.BlockSpec((8, 16), cst)
        out_shape = jax.ShapeDtypeStruct((8, 16), f32)
    else:
        out_spec = pl.BlockSpec((_NBC, dim), blk_n)
        out_shape = jax.ShapeDtypeStruct((n_nodes, dim), f32)
    return _PC(body, grid=grid, in_specs=in_specs, out_specs=out_spec,
               out_shape=out_shape)(*ins)


def _finalize(sums, cnt, gamma, beta):
    mean = sums[0, :16] / cnt
    var = sums[1, :16] / cnt - mean * mean
    a = gamma * lax.rsqrt(var + _EPS)
    c = beta - mean * a
    return a, c


def kernel(x, pos, knn_idx,
           W_first, b_first, gamma_first, beta_first,
           W_mid0, b_mid0, gamma_mid0, beta_mid0,
           W_last, b_last, gamma_last, beta_last,
           W_out, b_out):
    f32 = jnp.float32
    B, C, N = x.shape
    K = knn_idx.shape[-1]
    DIM = W_out.shape[0]
    M = B * N
    cnt = float(B * N * K)

    xt = x.transpose(0, 2, 1).reshape(M, C)
    post = jnp.pad(pos.transpose(0, 2, 1), ((0, 0), (0, 0), (0, 5)))
    post = post.reshape(M, 8)
    idx_flat = (knn_idx + (jnp.arange(B, dtype=jnp.int32) * N)[:, None, None])
    idx_flat = idx_flat.reshape(M * K)

    w1t = W_first[:, 3:].T
    wpt = jnp.pad(W_first[:, :3].T, ((0, 5), (0, 0)))
    w2t_full = W_mid0[:, 16:].T
    w3t_full = W_last[:, 32:].T
    wot_full = W_out[:, 48:].T
    bc = jnp.zeros((8, DIM), f32)
    bc = bc.at[0, :16].set(b_mid0).at[1, :16].set(b_last).at[2, :].set(b_out)

    w2a = W_mid0[:, :16].T
    w3c = W_last[:, :32].T
    wo = W_out[:, :48].T

    pT, c2T, c3T, coT = _node_tables_call(xt, post, w1t, wpt, w2t_full,
                                          w3t_full, wot_full, bc)

    g = _sc_gather(pT, idx_flat)

    def consts(rows):
        c = jnp.zeros((8, 16), f32)
        for i, r in enumerate(rows):
            c = c.at[i, :].set(r)
        return c

    s1 = _edge_pass(_stats1_body, [g, pT, consts([b_first])], M, K)
    a1, c1 = _finalize(s1, cnt, gamma_first, beta_first)

    s2 = _edge_pass(_stats2_body,
                    [g, pT, c2T, w2a, consts([b_first, a1, c1])], M, K)
    a2, c2 = _finalize(s2, cnt, gamma_mid0, beta_mid0)

    s3 = _edge_pass(_stats3_body,
                    [g, pT, c2T, c3T, w2a, w3c,
                     consts([b_first, a1, c1, a2, c2])], M, K)
    a3, c3 = _finalize(s3, cnt, gamma_last, beta_last)

    out = _edge_pass(_final_body,
                     [g, pT, c2T, c3T, coT, w2a, w3c, wo,
                      consts([b_first, a1, c1, a2, c2, a3, c3])],
                     M, K, dim=DIM)

    y = out.reshape(B, N, DIM).transpose(0, 2, 1)
    return (y, knn_idx)


# speedup vs baseline: 12.7885x; 1.1737x over previous
import functools

import jax
import jax.numpy as jnp
from jax import lax
from jax.experimental import pallas as pl
from jax.experimental.pallas import tpu as pltpu
from jax.experimental.pallas import tpu_sc as plsc

_PC = pl.pallas_call

_NBA = 1000
_NBC = 400
_EPS = 1e-5


def _expand_packed(t):
    nb, c = t.shape
    d = jnp.broadcast_to(t[:, None, :], (nb, 2, c)).reshape(2 * nb, c)
    return jnp.concatenate([d] * 8, axis=1)


def _proj_body(xt_ref, post_ref, w1t_ref, wpt_ref, w2t_ref, w3t_ref, wot_ref,
               bc_ref, p_ref, c2_ref, c3_ref, co_ref):
    xt = xt_ref[...]
    f32 = jnp.float32
    p_ref[...] = (jnp.dot(xt, w1t_ref[...], preferred_element_type=f32)
                  + jnp.dot(post_ref[...], wpt_ref[...],
                            preferred_element_type=f32))
    c2_ref[...] = (jnp.dot(xt, w2t_ref[...], preferred_element_type=f32)
                   + bc_ref[0:1, :16])
    c3_ref[...] = (jnp.dot(xt, w3t_ref[...], preferred_element_type=f32)
                   + bc_ref[1:2, :16])
    co_ref[...] = (jnp.dot(xt, wot_ref[...], preferred_element_type=f32)
                   + bc_ref[2:3, :])


def _node_tables_call(xt, post, w1t, wpt, w2t, w3t, wot, bc):
    m, c = xt.shape
    dim = wot.shape[1]
    grid = (m // _NBA,)
    blk = lambda i: (i, 0)
    cst = lambda i: (0, 0)
    f32 = jnp.float32
    return _PC(
        _proj_body,
        grid=grid,
        in_specs=[
            pl.BlockSpec((_NBA, c), blk),
            pl.BlockSpec((_NBA, 8), blk),
            pl.BlockSpec((c, 16), cst),
            pl.BlockSpec((8, 16), cst),
            pl.BlockSpec((c, 16), cst),
            pl.BlockSpec((c, 16), cst),
            pl.BlockSpec((c, dim), cst),
            pl.BlockSpec((8, dim), cst),
        ],
        out_specs=[
            pl.BlockSpec((_NBA, 16), blk),
            pl.BlockSpec((_NBA, 16), blk),
            pl.BlockSpec((_NBA, 16), blk),
            pl.BlockSpec((_NBA, dim), blk),
        ],
        out_shape=[
            jax.ShapeDtypeStruct((m, 16), f32),
            jax.ShapeDtypeStruct((m, 16), f32),
            jax.ShapeDtypeStruct((m, 16), f32),
            jax.ShapeDtypeStruct((m, dim), f32),
        ],
    )(xt, post, w1t, wpt, w2t, w3t, wot, bc)


def _sc_gather(table, idx_flat):
    e_total = idx_flat.shape[0]
    info = plsc.get_sparse_core_info()
    nc, ns = info.num_cores, info.num_subcores
    nw = nc * ns
    ew = e_total // nw
    ch = 80
    nch = ew // ch
    mesh = plsc.VectorSubcoreMesh(core_axis_name="c", subcore_axis_name="s")

    @functools.partial(
        pl.kernel,
        out_type=jax.ShapeDtypeStruct((e_total, 16), jnp.float32),
        mesh=mesh,
        scratch_types=[
            pltpu.VMEM((ch,), jnp.int32),
            pltpu.VMEM((ch, 16), jnp.float32),
            pltpu.SemaphoreType.DMA,
        ],
        compiler_params=pltpu.CompilerParams(use_tc_tiling_on_sc=False),
    )
    def k(table_hbm, idx_hbm, out_hbm, idx_v, rows_v, sem):
        wid = lax.axis_index("s") * nc + lax.axis_index("c")
        base = wid * ew

        def body(j, carry):
            off = base + j * ch
            pltpu.sync_copy(idx_hbm.at[pl.ds(off, ch)], idx_v)
            pltpu.async_copy(table_hbm.at[idx_v], rows_v, sem).wait()
            pltpu.sync_copy(rows_v, out_hbm.at[pl.ds(off, ch)])
            return carry

        lax.fori_loop(0, nch, body, 0)

    return k(table, idx_flat)


def _layer1(g_ref, p_ref, c_ref):
    return g_ref[...] - _expand_packed(p_ref[...]) + c_ref[0:1, :]


def _layer2(y1, c2_ref, bd2_ref, c_ref):
    y2p = (jnp.dot(y1, bd2_ref[...], preferred_element_type=jnp.float32)
           + _expand_packed(c2_ref[...]))
    return y2p


def _layer3(y1, y2, c3_ref, bd3a_ref, bd3b_ref, c_ref):
    return (jnp.dot(y2, bd3a_ref[...], preferred_element_type=jnp.float32)
            + jnp.dot(y1, bd3b_ref[...], preferred_element_type=jnp.float32)
            + _expand_packed(c3_ref[...]))


def _acc_stats(o_ref, y):
    @pl.when(pl.program_id(0) == 0)
    def _():
        o_ref[...] = jnp.zeros_like(o_ref)

    o_ref[0:1, :] += jnp.sum(y, axis=0, keepdims=True)
    o_ref[1:2, :] += jnp.sum(y * y, axis=0, keepdims=True)


def _stats1_body(g_ref, p_ref, c_ref, o_ref):
    _acc_stats(o_ref, _layer1(g_ref, p_ref, c_ref))


def _stats2_body(g_ref, p_ref, c2_ref, bd2_ref, c_ref, o_ref):
    y1p = _layer1(g_ref, p_ref, c_ref)
    y1 = jnp.maximum(c_ref[1:2, :] * y1p + c_ref[2:3, :], 0.0)
    _acc_stats(o_ref, _layer2(y1, c2_ref, bd2_ref, c_ref))


def _stats3_body(g_ref, p_ref, c2_ref, c3_ref, bd2_ref, bd3a_ref, bd3b_ref,
                 c_ref, o_ref):
    y1p = _layer1(g_ref, p_ref, c_ref)
    y1 = jnp.maximum(c_ref[1:2, :] * y1p + c_ref[2:3, :], 0.0)
    y2p = _layer2(y1, c2_ref, bd2_ref, c_ref)
    y2 = jnp.maximum(c_ref[3:4, :] * y2p + c_ref[4:5, :], 0.0)
    _acc_stats(o_ref, _layer3(y1, y2, c3_ref, bd3a_ref, bd3b_ref, c_ref))


def _final_body(g_ref, p_ref, c2_ref, c3_ref, co_ref, bd2_ref, bd3a_ref,
                bd3b_ref, wo_ref, c_ref, o_ref):
    nb = p_ref.shape[0]
    y1p = _layer1(g_ref, p_ref, c_ref)
    y1 = jnp.maximum(c_ref[1:2, :] * y1p + c_ref[2:3, :], 0.0)
    y2p = _layer2(y1, c2_ref, bd2_ref, c_ref)
    y2 = jnp.maximum(c_ref[3:4, :] * y2p + c_ref[4:5, :], 0.0)
    y3p = _layer3(y1, y2, c3_ref, bd3a_ref, bd3b_ref, c_ref)
    y3 = jnp.maximum(c_ref[5:6, :] * y3p + c_ref[6:7, :], 0.0)
    dim = wo_ref.shape[1]
    m = None
    for a in range(8):
        sl = slice(16 * a, 16 * (a + 1))
        ua = jnp.concatenate([y3[:, sl], y2[:, sl], y1[:, sl]], axis=1)
        oa = jnp.dot(ua, wo_ref[...], preferred_element_type=jnp.float32)
        oa3 = oa.reshape(nb, 2, dim)
        cand = jnp.maximum(oa3[:, 0, :], oa3[:, 1, :])
        m = cand if m is None else jnp.maximum(m, cand)
    o_ref[...] = m + co_ref[...]


def _edge_pass(body, ins, n_nodes, dim=None):
    r_total = 2 * n_nodes
    rblk = 2 * _NBC
    grid = (n_nodes // _NBC,)
    blk = lambda i: (i, 0)
    cst = lambda i: (0, 0)
    f32 = jnp.float32
    in_specs = []
    for a in ins:
        if a.shape[0] == r_total:
            in_specs.append(pl.BlockSpec((rblk, 128), blk))
        elif a.shape[0] == n_nodes:
            in_specs.append(pl.BlockSpec((_NBC, a.shape[1]), blk))
        else:
            in_specs.append(pl.BlockSpec(a.shape, cst))
    if dim is None:
        out_spec = pl.BlockSpec((8, 128), cst)
        out_shape = jax.ShapeDtypeStruct((8, 128), f32)
    else:
        out_spec = pl.BlockSpec((_NBC, dim), blk)
        out_shape = jax.ShapeDtypeStruct((n_nodes, dim), f32)
    return _PC(body, grid=grid, in_specs=in_specs, out_specs=out_spec,
               out_shape=out_shape)(*ins)


def _finalize(sums, cnt, gamma, beta):
    mean = sums[0].reshape(8, 16).sum(axis=0) / cnt
    var = sums[1].reshape(8, 16).sum(axis=0) / cnt - mean * mean
    a = gamma * lax.rsqrt(var + _EPS)
    c = beta - mean * a
    return a, c


def kernel(x, pos, knn_idx,
           W_first, b_first, gamma_first, beta_first,
           W_mid0, b_mid0, gamma_mid0, beta_mid0,
           W_last, b_last, gamma_last, beta_last,
           W_out, b_out):
    f32 = jnp.float32
    B, C, N = x.shape
    K = knn_idx.shape[-1]
    DIM = W_out.shape[0]
    M = B * N
    cnt = float(B * N * K)

    xt = x.transpose(0, 2, 1).reshape(M, C)
    post = jnp.pad(pos.transpose(0, 2, 1), ((0, 0), (0, 0), (0, 5)))
    post = post.reshape(M, 8)
    idx_flat = (knn_idx + (jnp.arange(B, dtype=jnp.int32) * N)[:, None, None])
    idx_flat = idx_flat.reshape(M * K)

    eye8 = jnp.eye(8, dtype=f32)
    w1t = W_first[:, 3:].T
    wpt = jnp.pad(W_first[:, :3].T, ((0, 5), (0, 0)))
    w2t_full = W_mid0[:, 16:].T
    w3t_full = W_last[:, 32:].T
    wot_full = W_out[:, 48:].T
    bc = jnp.zeros((8, DIM), f32)
    bc = bc.at[0, :16].set(b_mid0).at[1, :16].set(b_last).at[2, :].set(b_out)

    bd2 = jnp.kron(eye8, W_mid0[:, :16].T)
    bd3a = jnp.kron(eye8, W_last[:, :16].T)
    bd3b = jnp.kron(eye8, W_last[:, 16:32].T)
    wo = W_out[:, :48].T

    pT, c2T, c3T, coT = _node_tables_call(xt, post, w1t, wpt, w2t_full,
                                          w3t_full, wot_full, bc)

    g = _sc_gather(pT, idx_flat)
    gp = g.reshape(M * K // 8, 128)

    def consts(rows):
        c = jnp.zeros((8, 128), f32)
        for i, r in enumerate(rows):
            c = c.at[i, :].set(jnp.tile(r, 8))
        return c

    s1 = _edge_pass(_stats1_body, [gp, pT, consts([b_first])], M)
    a1, c1 = _finalize(s1, cnt, gamma_first, beta_first)

    s2 = _edge_pass(_stats2_body,
                    [gp, pT, c2T, bd2, consts([b_first, a1, c1])], M)
    a2, c2 = _finalize(s2, cnt, gamma_mid0, beta_mid0)

    s3 = _edge_pass(_stats3_body,
                    [gp, pT, c2T, c3T, bd2, bd3a, bd3b,
                     consts([b_first, a1, c1, a2, c2])], M)
    a3, c3 = _finalize(s3, cnt, gamma_last, beta_last)

    out = _edge_pass(_final_body,
                     [gp, pT, c2T, c3T, coT, bd2, bd3a, bd3b, wo,
                      consts([b_first, a1, c1, a2, c2, a3, c3])],
                     M, dim=DIM)

    y = out.reshape(B, N, DIM).transpose(0, 2, 1)
    return (y, knn_idx)


# speedup vs baseline: 17.8023x; 1.3921x over previous
import functools

import jax
import jax.numpy as jnp
from jax import lax
from jax.experimental import pallas as pl
from jax.experimental.pallas import tpu as pltpu
from jax.experimental.pallas import tpu_sc as plsc

_PC = pl.pallas_call

_NBA = 1000
_NBC = 400
_EPS = 1e-5


def _expand_packed(t, s):
    nb = t.shape[0]
    d = jnp.dot(t, s, preferred_element_type=jnp.float32)
    return jnp.broadcast_to(d[:, None, :], (nb, 2, 128)).reshape(2 * nb, 128)


def _proj_body(xt_ref, post_ref, w1t_ref, wpt_ref, w2t_ref, w3t_ref, wot_ref,
               bc_ref, p_ref, c2_ref, c3_ref, co_ref):
    xt = xt_ref[...]
    f32 = jnp.float32
    p_ref[...] = (jnp.dot(xt, w1t_ref[...], preferred_element_type=f32)
                  + jnp.dot(post_ref[...], wpt_ref[...],
                            preferred_element_type=f32))
    c2_ref[...] = (jnp.dot(xt, w2t_ref[...], preferred_element_type=f32)
                   + bc_ref[0:1, :16])
    c3_ref[...] = (jnp.dot(xt, w3t_ref[...], preferred_element_type=f32)
                   + bc_ref[1:2, :16])
    co_ref[...] = (jnp.dot(xt, wot_ref[...], preferred_element_type=f32)
                   + bc_ref[2:3, :])


def _node_tables_call(xt, post, w1t, wpt, w2t, w3t, wot, bc):
    m, c = xt.shape
    dim = wot.shape[1]
    grid = (m // _NBA,)
    blk = lambda i: (i, 0)
    cst = lambda i: (0, 0)
    f32 = jnp.float32
    return _PC(
        _proj_body,
        grid=grid,
        in_specs=[
            pl.BlockSpec((_NBA, c), blk),
            pl.BlockSpec((_NBA, 8), blk),
            pl.BlockSpec((c, 16), cst),
            pl.BlockSpec((8, 16), cst),
            pl.BlockSpec((c, 16), cst),
            pl.BlockSpec((c, 16), cst),
            pl.BlockSpec((c, dim), cst),
            pl.BlockSpec((8, dim), cst),
        ],
        out_specs=[
            pl.BlockSpec((_NBA, 16), blk),
            pl.BlockSpec((_NBA, 16), blk),
            pl.BlockSpec((_NBA, 16), blk),
            pl.BlockSpec((_NBA, dim), blk),
        ],
        out_shape=[
            jax.ShapeDtypeStruct((m, 16), f32),
            jax.ShapeDtypeStruct((m, 16), f32),
            jax.ShapeDtypeStruct((m, 16), f32),
            jax.ShapeDtypeStruct((m, dim), f32),
        ],
    )(xt, post, w1t, wpt, w2t, w3t, wot, bc)


def _sc_gather(table, idx_flat):
    e_total = idx_flat.shape[0]
    info = plsc.get_sparse_core_info()
    nc, ns = info.num_cores, info.num_subcores
    nw = nc * ns
    ew = e_total // nw
    ch = 80
    nch = ew // ch
    mesh = plsc.VectorSubcoreMesh(core_axis_name="c", subcore_axis_name="s")

    @functools.partial(
        pl.kernel,
        out_type=jax.ShapeDtypeStruct((e_total, 16), jnp.float32),
        mesh=mesh,
        scratch_types=[
            pltpu.VMEM((ch,), jnp.int32),
            pltpu.VMEM((ch, 16), jnp.float32),
            pltpu.SemaphoreType.DMA,
        ],
        compiler_params=pltpu.CompilerParams(use_tc_tiling_on_sc=False),
    )
    def k(table_hbm, idx_hbm, out_hbm, idx_v, rows_v, sem):
        wid = lax.axis_index("s") * nc + lax.axis_index("c")
        base = wid * ew

        def body(j, carry):
            off = base + j * ch
            pltpu.sync_copy(idx_hbm.at[pl.ds(off, ch)], idx_v)
            pltpu.async_copy(table_hbm.at[idx_v], rows_v, sem).wait()
            pltpu.sync_copy(rows_v, out_hbm.at[pl.ds(off, ch)])
            return carry

        lax.fori_loop(0, nch, body, 0)

    return k(table, idx_flat)


def _layer1(g_ref, p_ref, c_ref, s):
    return g_ref[...] - _expand_packed(p_ref[...], s) + c_ref[0:1, :]


def _layer2(y1, c2_ref, bd2_ref, c_ref, s):
    y2p = (jnp.dot(y1, bd2_ref[...], preferred_element_type=jnp.float32)
           + _expand_packed(c2_ref[...], s))
    return y2p


def _layer3(y1, y2, c3_ref, bd3a_ref, bd3b_ref, c_ref, s):
    return (jnp.dot(y2, bd3a_ref[...], preferred_element_type=jnp.float32)
            + jnp.dot(y1, bd3b_ref[...], preferred_element_type=jnp.float32)
            + _expand_packed(c3_ref[...], s))


def _acc_stats(o_ref, y):
    @pl.when(pl.program_id(0) == 0)
    def _():
        o_ref[...] = jnp.zeros_like(o_ref)

    o_ref[0:1, :] += jnp.sum(y, axis=0, keepdims=True)
    o_ref[1:2, :] += jnp.sum(y * y, axis=0, keepdims=True)


def _stats1_body(g_ref, p_ref, s_ref, c_ref, o_ref):
    _acc_stats(o_ref, _layer1(g_ref, p_ref, c_ref, s_ref[...]))


def _stats2_body(g_ref, p_ref, c2_ref, bd2_ref, s_ref, c_ref, o_ref):
    s = s_ref[...]
    y1p = _layer1(g_ref, p_ref, c_ref, s)
    y1 = jnp.maximum(c_ref[1:2, :] * y1p + c_ref[2:3, :], 0.0)
    _acc_stats(o_ref, _layer2(y1, c2_ref, bd2_ref, c_ref, s))


def _stats3_body(g_ref, p_ref, c2_ref, c3_ref, bd2_ref, bd3a_ref, bd3b_ref,
                 s_ref, c_ref, o_ref):
    s = s_ref[...]
    y1p = _layer1(g_ref, p_ref, c_ref, s)
    y1 = jnp.maximum(c_ref[1:2, :] * y1p + c_ref[2:3, :], 0.0)
    y2p = _layer2(y1, c2_ref, bd2_ref, c_ref, s)
    y2 = jnp.maximum(c_ref[3:4, :] * y2p + c_ref[4:5, :], 0.0)
    _acc_stats(o_ref, _layer3(y1, y2, c3_ref, bd3a_ref, bd3b_ref, c_ref, s))


def _final_body(g_ref, p_ref, c2_ref, c3_ref, co_ref, bd2_ref, bd3a_ref,
                bd3b_ref, wo_ref, s_ref, c_ref, o_ref):
    nb = p_ref.shape[0]
    s = s_ref[...]
    y1p = _layer1(g_ref, p_ref, c_ref, s)
    y1 = jnp.maximum(c_ref[1:2, :] * y1p + c_ref[2:3, :], 0.0)
    y2p = _layer2(y1, c2_ref, bd2_ref, c_ref, s)
    y2 = jnp.maximum(c_ref[3:4, :] * y2p + c_ref[4:5, :], 0.0)
    y3p = _layer3(y1, y2, c3_ref, bd3a_ref, bd3b_ref, c_ref, s)
    y3 = jnp.maximum(c_ref[5:6, :] * y3p + c_ref[6:7, :], 0.0)
    dim = wo_ref.shape[1]
    m = None
    for a in range(8):
        sl = slice(16 * a, 16 * (a + 1))
        ua = jnp.concatenate([y3[:, sl], y2[:, sl], y1[:, sl]], axis=1)
        oa = jnp.dot(ua, wo_ref[...], preferred_element_type=jnp.float32)
        oa3 = oa.reshape(nb, 2, dim)
        cand = jnp.maximum(oa3[:, 0, :], oa3[:, 1, :])
        m = cand if m is None else jnp.maximum(m, cand)
    o_ref[...] = m + co_ref[...]


def _edge_pass(body, ins, n_nodes, dim=None):
    r_total = 2 * n_nodes
    rblk = 2 * _NBC
    grid = (n_nodes // _NBC,)
    blk = lambda i: (i, 0)
    cst = lambda i: (0, 0)
    f32 = jnp.float32
    in_specs = []
    for a in ins:
        if a.shape[0] == r_total:
            in_specs.append(pl.BlockSpec((rblk, 128), blk))
        elif a.shape[0] == n_nodes:
            in_specs.append(pl.BlockSpec((_NBC, a.shape[1]), blk))
        else:
            in_specs.append(pl.BlockSpec(a.shape, cst))
    if dim is None:
        out_spec = pl.BlockSpec((8, 128), cst)
        out_shape = jax.ShapeDtypeStruct((8, 128), f32)
    else:
        out_spec = pl.BlockSpec((_NBC, dim), blk)
        out_shape = jax.ShapeDtypeStruct((n_nodes, dim), f32)
    return _PC(body, grid=grid, in_specs=in_specs, out_specs=out_spec,
               out_shape=out_shape)(*ins)


def _finalize(sums, cnt, gamma, beta):
    mean = sums[0].reshape(8, 16).sum(axis=0) / cnt
    var = sums[1].reshape(8, 16).sum(axis=0) / cnt - mean * mean
    a = gamma * lax.rsqrt(var + _EPS)
    c = beta - mean * a
    return a, c


def kernel(x, pos, knn_idx,
           W_first, b_first, gamma_first, beta_first,
           W_mid0, b_mid0, gamma_mid0, beta_mid0,
           W_last, b_last, gamma_last, beta_last,
           W_out, b_out):
    f32 = jnp.float32
    B, C, N = x.shape
    K = knn_idx.shape[-1]
    DIM = W_out.shape[0]
    M = B * N
    cnt = float(B * N * K)

    xt = x.transpose(0, 2, 1).reshape(M, C)
    post = jnp.pad(pos.transpose(0, 2, 1), ((0, 0), (0, 0), (0, 5)))
    post = post.reshape(M, 8)
    idx_flat = (knn_idx + (jnp.arange(B, dtype=jnp.int32) * N)[:, None, None])
    idx_flat = idx_flat.reshape(M * K)

    eye8 = jnp.eye(8, dtype=f32)
    w1t = W_first[:, 3:].T
    wpt = jnp.pad(W_first[:, :3].T, ((0, 5), (0, 0)))
    w2t_full = W_mid0[:, 16:].T
    w3t_full = W_last[:, 32:].T
    wot_full = W_out[:, 48:].T
    bc = jnp.zeros((8, DIM), f32)
    bc = bc.at[0, :16].set(b_mid0).at[1, :16].set(b_last).at[2, :].set(b_out)

    bd2 = jnp.kron(eye8, W_mid0[:, :16].T)
    bd3a = jnp.kron(eye8, W_last[:, :16].T)
    bd3b = jnp.kron(eye8, W_last[:, 16:32].T)
    wo = W_out[:, :48].T

    pT, c2T, c3T, coT = _node_tables_call(xt, post, w1t, wpt, w2t_full,
                                          w3t_full, wot_full, bc)

    g = _sc_gather(pT, idx_flat)
    gp = g.reshape(M * K // 8, 128)

    def consts(rows):
        c = jnp.zeros((8, 128), f32)
        for i, r in enumerate(rows):
            c = c.at[i, :].set(jnp.tile(r, 8))
        return c

    sel = jnp.tile(jnp.eye(16, dtype=f32), (1, 8))

    s1 = _edge_pass(_stats1_body, [gp, pT, sel, consts([b_first])], M)
    a1, c1 = _finalize(s1, cnt, gamma_first, beta_first)

    s2 = _edge_pass(_stats2_body,
                    [gp, pT, c2T, bd2, sel, consts([b_first, a1, c1])], M)
    a2, c2 = _finalize(s2, cnt, gamma_mid0, beta_mid0)

    s3 = _edge_pass(_stats3_body,
                    [gp, pT, c2T, c3T, bd2, bd3a, bd3b, sel,
                     consts([b_first, a1, c1, a2, c2])], M)
    a3, c3 = _finalize(s3, cnt, gamma_last, beta_last)

    out = _edge_pass(_final_body,
                     [gp, pT, c2T, c3T, coT, bd2, bd3a, bd3b, wo, sel,
                      consts([b_first, a1, c1, a2, c2, a3, c3])],
                     M, dim=DIM)

    y = out.reshape(B, N, DIM).transpose(0, 2, 1)
    return (y, knn_idx)


# speedup vs baseline: 22.0421x; 1.2382x over previous
import functools

import jax
import jax.numpy as jnp
from jax import lax
from jax.experimental import pallas as pl
from jax.experimental.pallas import tpu as pltpu
from jax.experimental.pallas import tpu_sc as plsc

_PC = pl.pallas_call

_NBA = 1000
_NBC = 400
_EPS = 1e-5


def _expand_packed(t, s):
    nb = t.shape[0]
    d = jnp.dot(t, s, preferred_element_type=jnp.float32)
    return jnp.broadcast_to(d[:, None, :], (nb, 2, 128)).reshape(2 * nb, 128)


def _proj_body(xt_ref, post_ref, w1t_ref, wpt_ref, w2t_ref, w3t_ref, wot_ref,
               bc_ref, p_ref, c2_ref, c3_ref, co_ref):
    xt = xt_ref[...]
    f32 = jnp.float32
    p_ref[...] = (jnp.dot(xt, w1t_ref[...], preferred_element_type=f32)
                  + jnp.dot(post_ref[...], wpt_ref[...],
                            preferred_element_type=f32))
    c2_ref[...] = (jnp.dot(xt, w2t_ref[...], preferred_element_type=f32)
                   + bc_ref[0:1, :16])
    c3_ref[...] = (jnp.dot(xt, w3t_ref[...], preferred_element_type=f32)
                   + bc_ref[1:2, :16])
    co_ref[...] = (jnp.dot(xt, wot_ref[...], preferred_element_type=f32)
                   + bc_ref[2:3, :])


def _node_tables_call(xt, post, w1t, wpt, w2t, w3t, wot, bc):
    m, c = xt.shape
    dim = wot.shape[1]
    grid = (m // _NBA,)
    blk = lambda i: (i, 0)
    cst = lambda i: (0, 0)
    f32 = jnp.float32
    return _PC(
        _proj_body,
        grid=grid,
        in_specs=[
            pl.BlockSpec((_NBA, c), blk),
            pl.BlockSpec((_NBA, 8), blk),
            pl.BlockSpec((c, 16), cst),
            pl.BlockSpec((8, 16), cst),
            pl.BlockSpec((c, 16), cst),
            pl.BlockSpec((c, 16), cst),
            pl.BlockSpec((c, dim), cst),
            pl.BlockSpec((8, dim), cst),
        ],
        out_specs=[
            pl.BlockSpec((_NBA, 16), blk),
            pl.BlockSpec((_NBA, 16), blk),
            pl.BlockSpec((_NBA, 16), blk),
            pl.BlockSpec((_NBA, dim), blk),
        ],
        out_shape=[
            jax.ShapeDtypeStruct((m, 16), f32),
            jax.ShapeDtypeStruct((m, 16), f32),
            jax.ShapeDtypeStruct((m, 16), f32),
            jax.ShapeDtypeStruct((m, dim), f32),
        ],
    )(xt, post, w1t, wpt, w2t, w3t, wot, bc)


def _sc_gather(table, idx_flat):
    e_total = idx_flat.shape[0]
    info = plsc.get_sparse_core_info()
    nc, ns = info.num_cores, info.num_subcores
    nw = nc * ns
    ew = e_total // nw
    ch = 80
    nch = ew // ch
    mesh = plsc.VectorSubcoreMesh(core_axis_name="c", subcore_axis_name="s")

    @functools.partial(
        pl.kernel,
        out_type=jax.ShapeDtypeStruct((e_total // 8, 128), jnp.float32),
        mesh=mesh,
        scratch_types=[
            pltpu.VMEM((ew,), jnp.int32),
            pltpu.VMEM((ch, 16), jnp.float32),
            pltpu.VMEM((ch, 16), jnp.float32),
            pltpu.VMEM((ch // 8, 128), jnp.float32),
            pltpu.SemaphoreType.DMA,
            pltpu.SemaphoreType.DMA,
        ],
        compiler_params=pltpu.CompilerParams(use_tc_tiling_on_sc=False),
    )
    def k(table_hbm, idx_hbm, out_hbm, idx_v, r0, r1, pk, s0, s1):
        wid = lax.axis_index("s") * nc + lax.axis_index("c")
        base = wid * ew
        obase = base // 8
        pltpu.sync_copy(idx_hbm.at[pl.ds(base, ew)], idx_v)
        pltpu.async_copy(table_hbm.at[idx_v.at[pl.ds(0, ch)]], r0, s0)

        def body(j, carry):
            nxt = j + 1

            @pl.when(nxt < nch)
            def _():
                sl = idx_v.at[pl.ds(nxt * ch, ch)]

                @pl.when(nxt % 2 == 0)
                def _():
                    pltpu.async_copy(table_hbm.at[sl], r0, s0)

                @pl.when(nxt % 2 == 1)
                def _():
                    pltpu.async_copy(table_hbm.at[sl], r1, s1)

            dst = out_hbm.at[pl.ds(obase + j * (ch // 8), ch // 8)]

            def drain(rbuf, sem):
                pltpu.make_async_copy(
                    table_hbm.at[idx_v.at[pl.ds(j * ch, ch)]], rbuf, sem
                ).wait()
                for r in range(ch // 8):
                    for c in range(8):
                        pk[r, pl.ds(16 * c, 16)] = rbuf[8 * r + c, :]
                pltpu.sync_copy(pk, dst)

            @pl.when(j % 2 == 0)
            def _():
                drain(r0, s0)

            @pl.when(j % 2 == 1)
            def _():
                drain(r1, s1)

            return carry

        lax.fori_loop(0, nch, body, 0)

    return k(table, idx_flat)


def _layer1(g_ref, p_ref, c_ref, s):
    return g_ref[...] - _expand_packed(p_ref[...], s) + c_ref[0:1, :]


def _layer2(y1, c2_ref, bd2_ref, c_ref, s):
    y2p = (jnp.dot(y1, bd2_ref[...], preferred_element_type=jnp.float32)
           + _expand_packed(c2_ref[...], s))
    return y2p


def _layer3(y1, y2, c3_ref, bd3a_ref, bd3b_ref, c_ref, s):
    return (jnp.dot(y2, bd3a_ref[...], preferred_element_type=jnp.float32)
            + jnp.dot(y1, bd3b_ref[...], preferred_element_type=jnp.float32)
            + _expand_packed(c3_ref[...], s))


def _acc_stats(o_ref, y):
    @pl.when(pl.program_id(0) == 0)
    def _():
        o_ref[...] = jnp.zeros_like(o_ref)

    o_ref[0:1, :] += jnp.sum(y, axis=0, keepdims=True)
    o_ref[1:2, :] += jnp.sum(y * y, axis=0, keepdims=True)


def _stats1_body(g_ref, p_ref, s_ref, c_ref, o_ref):
    _acc_stats(o_ref, _layer1(g_ref, p_ref, c_ref, s_ref[...]))


def _stats2_body(g_ref, p_ref, c2_ref, bd2_ref, s_ref, c_ref, o_ref):
    s = s_ref[...]
    y1p = _layer1(g_ref, p_ref, c_ref, s)
    y1 = jnp.maximum(c_ref[1:2, :] * y1p + c_ref[2:3, :], 0.0)
    _acc_stats(o_ref, _layer2(y1, c2_ref, bd2_ref, c_ref, s))


def _stats3_body(g_ref, p_ref, c2_ref, c3_ref, bd2_ref, bd3a_ref, bd3b_ref,
                 s_ref, c_ref, o_ref):
    s = s_ref[...]
    y1p = _layer1(g_ref, p_ref, c_ref, s)
    y1 = jnp.maximum(c_ref[1:2, :] * y1p + c_ref[2:3, :], 0.0)
    y2p = _layer2(y1, c2_ref, bd2_ref, c_ref, s)
    y2 = jnp.maximum(c_ref[3:4, :] * y2p + c_ref[4:5, :], 0.0)
    _acc_stats(o_ref, _layer3(y1, y2, c3_ref, bd3a_ref, bd3b_ref, c_ref, s))


def _final_body(g_ref, p_ref, c2_ref, c3_ref, co_ref, bd2_ref, bd3a_ref,
                bd3b_ref, wo_ref, s_ref, c_ref, o_ref):
    nb = p_ref.shape[0]
    s = s_ref[...]
    y1p = _layer1(g_ref, p_ref, c_ref, s)
    y1 = jnp.maximum(c_ref[1:2, :] * y1p + c_ref[2:3, :], 0.0)
    y2p = _layer2(y1, c2_ref, bd2_ref, c_ref, s)
    y2 = jnp.maximum(c_ref[3:4, :] * y2p + c_ref[4:5, :], 0.0)
    y3p = _layer3(y1, y2, c3_ref, bd3a_ref, bd3b_ref, c_ref, s)
    y3 = jnp.maximum(c_ref[5:6, :] * y3p + c_ref[6:7, :], 0.0)
    dim = wo_ref.shape[1]
    m = None
    for a in range(8):
        sl = slice(16 * a, 16 * (a + 1))
        ua = jnp.concatenate([y3[:, sl], y2[:, sl], y1[:, sl]], axis=1)
        oa = jnp.dot(ua, wo_ref[...], preferred_element_type=jnp.float32)
        m = oa if m is None else jnp.maximum(m, oa)
    m3 = m.reshape(nb, 2, dim)
    o_ref[...] = jnp.maximum(m3[:, 0, :], m3[:, 1, :]) + co_ref[...]


def _edge_pass(body, ins, n_nodes, dim=None):
    r_total = 2 * n_nodes
    rblk = 2 * _NBC
    grid = (n_nodes // _NBC,)
    blk = lambda i: (i, 0)
    cst = lambda i: (0, 0)
    f32 = jnp.float32
    in_specs = []
    for a in ins:
        if a.shape[0] == r_total:
            in_specs.append(pl.BlockSpec((rblk, 128), blk))
        elif a.shape[0] == n_nodes:
            in_specs.append(pl.BlockSpec((_NBC, a.shape[1]), blk))
        else:
            in_specs.append(pl.BlockSpec(a.shape, cst))
    if dim is None:
        out_spec = pl.BlockSpec((8, 128), cst)
        out_shape = jax.ShapeDtypeStruct((8, 128), f32)
    else:
        out_spec = pl.BlockSpec((_NBC, dim), blk)
        out_shape = jax.ShapeDtypeStruct((n_nodes, dim), f32)
    return _PC(body, grid=grid, in_specs=in_specs, out_specs=out_spec,
               out_shape=out_shape)(*ins)


def _finalize(sums, cnt, gamma, beta):
    mean = sums[0].reshape(8, 16).sum(axis=0) / cnt
    var = sums[1].reshape(8, 16).sum(axis=0) / cnt - mean * mean
    a = gamma * lax.rsqrt(var + _EPS)
    c = beta - mean * a
    return a, c


def kernel(x, pos, knn_idx,
           W_first, b_first, gamma_first, beta_first,
           W_mid0, b_mid0, gamma_mid0, beta_mid0,
           W_last, b_last, gamma_last, beta_last,
           W_out, b_out):
    f32 = jnp.float32
    B, C, N = x.shape
    K = knn_idx.shape[-1]
    DIM = W_out.shape[0]
    M = B * N
    cnt = float(B * N * K)

    xt = x.transpose(0, 2, 1).reshape(M, C)
    post = jnp.pad(pos.transpose(0, 2, 1), ((0, 0), (0, 0), (0, 5)))
    post = post.reshape(M, 8)
    idx_flat = (knn_idx + (jnp.arange(B, dtype=jnp.int32) * N)[:, None, None])
    idx_flat = idx_flat.reshape(M * K)

    eye8 = jnp.eye(8, dtype=f32)
    w1t = W_first[:, 3:].T
    wpt = jnp.pad(W_first[:, :3].T, ((0, 5), (0, 0)))
    w2t_full = W_mid0[:, 16:].T
    w3t_full = W_last[:, 32:].T
    wot_full = W_out[:, 48:].T
    bc = jnp.zeros((8, DIM), f32)
    bc = bc.at[0, :16].set(b_mid0).at[1, :16].set(b_last).at[2, :].set(b_out)

    bd2 = jnp.kron(eye8, W_mid0[:, :16].T)
    bd3a = jnp.kron(eye8, W_last[:, :16].T)
    bd3b = jnp.kron(eye8, W_last[:, 16:32].T)
    wo = W_out[:, :48].T

    pT, c2T, c3T, coT = _node_tables_call(xt, post, w1t, wpt, w2t_full,
                                          w3t_full, wot_full, bc)

    gp = _sc_gather(pT, idx_flat)

    def consts(rows):
        c = jnp.zeros((8, 128), f32)
        for i, r in enumerate(rows):
            c = c.at[i, :].set(jnp.tile(r, 8))
        return c

    sel = jnp.tile(jnp.eye(16, dtype=f32), (1, 8))

    s1 = _edge_pass(_stats1_body, [gp, pT, sel, consts([b_first])], M)
    a1, c1 = _finalize(s1, cnt, gamma_first, beta_first)

    s2 = _edge_pass(_stats2_body,
                    [gp, pT, c2T, bd2, sel, consts([b_first, a1, c1])], M)
    a2, c2 = _finalize(s2, cnt, gamma_mid0, beta_mid0)

    s3 = _edge_pass(_stats3_body,
                    [gp, pT, c2T, c3T, bd2, bd3a, bd3b, sel,
                     consts([b_first, a1, c1, a2, c2])], M)
    a3, c3 = _finalize(s3, cnt, gamma_last, beta_last)

    out = _edge_pass(_final_body,
                     [gp, pT, c2T, c3T, coT, bd2, bd3a, bd3b, wo, sel,
                      consts([b_first, a1, c1, a2, c2, a3, c3])],
                     M, dim=DIM)

    y = out.reshape(B, N, DIM).transpose(0, 2, 1)
    return (y, knn_idx)
